# baseline (speedup 1.0000x reference)
import functools

import jax
import jax.numpy as jnp
from jax import lax
from jax.experimental import pallas as pl
from jax.experimental.pallas import tpu as pltpu
from jax.experimental.pallas import tpu_sc as plsc

VOCAB_N = 1000000
D_MODEL = 64
T_LEN = 200
B_LEN = 4096
NUM_TILES = 32
B_BLK = B_LEN // NUM_TILES
TOK_PER_TILE = T_LEN * B_BLK
NRING = 2

_mesh = plsc.VectorSubcoreMesh(core_axis_name="c", subcore_axis_name="s")


@functools.partial(
    pl.kernel,
    mesh=_mesh,
    out_type=jax.ShapeDtypeStruct((T_LEN, D_MODEL, B_LEN), jnp.float32),
    compiler_params=pltpu.CompilerParams(needs_layout_passes=False,
                                         use_tc_tiling_on_sc=False),
    scratch_types=[pltpu.VMEM((TOK_PER_TILE,), jnp.int32)]
    + [pltpu.VMEM((2 * B_BLK, D_MODEL), jnp.float32)] * NRING
    + [pltpu.VMEM((D_MODEL, B_BLK + 1), jnp.float32)] * 2
    + [pltpu.SemaphoreType.DMA] * (NRING + 2),
)
def _gather_kernel(x_hbm, table_hbm, out_hbm, idx1, *bufs):
    gbufs = bufs[:NRING]
    tbufs = bufs[NRING:NRING + 2]
    gsems = bufs[NRING + 2:2 * NRING + 2]
    wsems = bufs[2 * NRING + 2:]
    cid = lax.axis_index("c")
    sid = lax.axis_index("s")
    wid = cid * 16 + sid
    tok0 = pl.multiple_of(wid * TOK_PER_TILE, 128)
    b0 = pl.multiple_of(wid * B_BLK, 128)

    pltpu.sync_copy(x_hbm.at[pl.ds(tok0, TOK_PER_TILE)], idx1)

    def gather_copy(tp, g):
        sl = idx1.at[pl.ds(pl.multiple_of(tp * 2 * B_BLK, 128), 2 * B_BLK)]
        return pltpu.make_async_copy(table_hbm.at[sl], gbufs[g], gsems[g])

    def write_copy(t, p):
        return pltpu.make_async_copy(
            tbufs[p].at[:, pl.ds(0, B_BLK)],
            out_hbm.at[t, :, pl.ds(b0, B_BLK)], wsems[p])

    lanes16 = lax.iota(jnp.int32, 16)
    drows = [lanes16 + (16 * j) for j in range(4)]
    zero16 = lanes16 * 0

    def extract(g, h):
        gbuf = gbufs[g]
        tbuf = tbufs[h]
        for tok in range(B_BLK):
            cols = zero16 + tok
            for j in range(4):
                val = gbuf[h * B_BLK + tok, pl.ds(16 * j, 16)]
                plsc.store_scatter(tbuf, [drows[j], cols], val)

    gather_copy(0, 0).start()
    gather_copy(1, 1).start()

    def body(i, carry):
        for g in range(2):
            tp = 2 * i + g
            gather_copy(tp, g).wait()
            for h in range(2):
                t = 2 * tp + h

                @pl.when(t >= 2)
                def _(t=t, h=h):
                    write_copy(t - 2, h).wait()

                extract(g, h)
                write_copy(t, h).start()

            @pl.when(tp + 2 < T_LEN // 2)
            def _(tp=tp, g=g):
                gather_copy(tp + 2, g).start()
        return carry

    lax.fori_loop(0, T_LEN // 4, body, 0)

    for tt in (T_LEN - 2, T_LEN - 1):
        write_copy(tt, tt % 2).wait()


def kernel(x, table):
    xr = x.T.reshape(T_LEN, NUM_TILES, B_BLK).transpose(1, 0, 2).reshape(-1)
    out3 = _gather_kernel(xr.astype(jnp.int32), table)
    return out3.transpose(2, 0, 1)


# speedup vs baseline: 1.1029x; 1.1029x over previous
import functools

import jax
import jax.numpy as jnp
from jax import lax
from jax.experimental import pallas as pl
from jax.experimental.pallas import tpu as pltpu
from jax.experimental.pallas import tpu_sc as plsc

D_MODEL = 64
N_TOKENS = 4096 * 200
NUM_CORES = 2
NUM_SUBCORES = 16
NUM_WORKERS = NUM_CORES * NUM_SUBCORES
ROWS_PER_WORKER = N_TOKENS // NUM_WORKERS
CHUNK = 400
NBUF = 4
STEPS = ROWS_PER_WORKER // CHUNK
GROUPS = STEPS // NBUF

_mesh = plsc.VectorSubcoreMesh(core_axis_name="c", subcore_axis_name="s")


@functools.partial(
    pl.kernel,
    mesh=_mesh,
    out_type=jax.ShapeDtypeStruct((N_TOKENS, D_MODEL), jnp.float32),
    compiler_params=pltpu.CompilerParams(use_tc_tiling_on_sc=False),
    scratch_types=[
        pltpu.VMEM((ROWS_PER_WORKER,), jnp.int32),
        pltpu.VMEM((NBUF, CHUNK, D_MODEL), jnp.float32),
    ]
    + [pltpu.SemaphoreType.DMA] * (2 * NBUF),
)
def _gather_kernel(idx_hbm, table_hbm, out_hbm, idx_v, rows_v, *sems):
    gsems = sems[:NBUF]
    ssems = sems[NBUF:]
    wid = lax.axis_index("s") * NUM_CORES + lax.axis_index("c")
    base = wid * ROWS_PER_WORKER

    pltpu.sync_copy(idx_hbm.at[pl.ds(base, ROWS_PER_WORKER)], idx_v)

    def idx_slice(g):
        return idx_v.at[pl.ds(g * CHUNK, CHUNK)]

    def gather_start(g, b):
        pltpu.async_copy(table_hbm.at[idx_slice(g)], rows_v.at[b], gsems[b])

    def gather_wait(g, b):
        pltpu.make_async_copy(table_hbm.at[idx_slice(g)], rows_v.at[b],
                              gsems[b]).wait()

    def out_slice(g):
        return out_hbm.at[pl.ds(base + g * CHUNK, CHUNK)]

    def store_start(g, b):
        pltpu.async_copy(rows_v.at[b], out_slice(g), ssems[b])

    def store_wait(g, b):
        pltpu.make_async_copy(rows_v.at[b], out_slice(g), ssems[b]).wait()

    for k in range(NBUF - 1):
        gather_start(k, k)
    gather_start(NBUF - 1, NBUF - 1)
    gather_wait(0, 0)
    store_start(0, 0)
    for g in range(1, NBUF):
        store_wait(g - 1, g - 1)
        gather_start(g + NBUF - 1, (g + NBUF - 1) % NBUF)
        gather_wait(g, g)
        store_start(g, g)

    def body(i, carry):
        for b in range(NBUF):
            g = i * NBUF + b
            store_wait(g - 1, (b - 1) % NBUF)
            gather_start(g + NBUF - 1, (b - 1) % NBUF)
            gather_wait(g, b)
            store_start(g, b)
        return carry

    lax.fori_loop(1, GROUPS - 1, body, 0)

    for b in range(NBUF):
        g = (GROUPS - 1) * NBUF + b
        if g + NBUF - 1 < STEPS:
            store_wait(g - 1, (g - 1) % NBUF)
            gather_start(g + NBUF - 1, (g + NBUF - 1) % NBUF)
        gather_wait(g, b)
        store_start(g, b)
    for b in range(NBUF):
        g = (GROUPS - 1) * NBUF + b
        store_wait(g, b)


def kernel(x, table):
    flat = x.reshape(-1).astype(jnp.int32)
    out = _gather_kernel(flat, table)
    return out.reshape(x.shape + (table.shape[1],))
